# baseline (device time: 36633 ns/iter reference)
import jax
import jax.numpy as jnp
from jax import lax
from jax.experimental import pallas as pl
from jax.experimental.pallas import tpu as pltpu

B, S, H, D = 4, 512, 8, 64
K = H * D
N = 1024
SH = S // 2


def kernel(O, Wo):
    O2 = O.reshape(B * S, K)

    def body(o_hbm, w_hbm, out_hbm,
             w_vmem, o_vmem, out_vmem,
             send_wo, recv_wo, send_o, recv_o,
             in_sems, out_sems, wo_sems, o_sems):
        my_x = lax.axis_index("x")
        my_y = lax.axis_index("y")
        my_z = lax.axis_index("z")
        other = 1 - my_x
        nbr = (other, my_y, my_z)

        w_in = pltpu.make_async_copy(w_hbm, w_vmem, in_sems.at[0])
        w_in.start()
        o_in = pltpu.make_async_copy(o_hbm, o_vmem, in_sems.at[1])
        o_in.start()

        barrier = pltpu.get_barrier_semaphore()
        pl.semaphore_signal(
            barrier, inc=1,
            device_id=nbr, device_id_type=pl.DeviceIdType.MESH,
        )
        pl.semaphore_wait(barrier, 1)

        w_in.wait()
        w = w_vmem[...].astype(jnp.bfloat16)
        send_wo[...] = w
        wo_rdma = pltpu.make_async_remote_copy(
            src_ref=send_wo, dst_ref=recv_wo,
            send_sem=wo_sems.at[0], recv_sem=wo_sems.at[1],
            device_id=nbr, device_id_type=pl.DeviceIdType.MESH,
        )
        wo_rdma.start()

        o_in.wait()
        o_rdmas = []
        for b in range(B):
            rows = o_vmem[pl.ds(b * S + other * SH, SH), :]
            send_o[pl.ds(b * SH, SH), :] = rows.astype(jnp.bfloat16)
            rdma = pltpu.make_async_remote_copy(
                src_ref=send_o.at[pl.ds(b * SH, SH), :],
                dst_ref=recv_o.at[pl.ds(b * SH, SH), :],
                send_sem=o_sems.at[0, b], recv_sem=o_sems.at[1, b],
                device_id=nbr, device_id_type=pl.DeviceIdType.MESH,
            )
            rdma.start()
            o_rdmas.append(rdma)

        for b in range(B):
            rows = o_vmem[pl.ds(b * S + my_x * SH, SH), :]
            acc = jnp.dot(rows.astype(jnp.bfloat16), w,
                          preferred_element_type=jnp.float32)
            out_vmem[b, :, :] = acc

        wo_rdma.wait_recv()
        out_copies = []
        for b in range(B):
            o_rdmas[b].wait_recv()
            acc = jnp.dot(recv_o[pl.ds(b * SH, SH), :], recv_wo[...],
                          preferred_element_type=jnp.float32)
            out_vmem[b, :, :] += acc
            cp = pltpu.make_async_copy(
                out_vmem.at[b], out_hbm.at[b], out_sems.at[b])
            cp.start()
            out_copies.append(cp)

        for cp in out_copies:
            cp.wait()
        wo_rdma.wait_send()
        for b in range(B):
            o_rdmas[b].wait_send()

    return pl.pallas_call(
        body,
        out_shape=jax.ShapeDtypeStruct((B, SH, N), jnp.float32),
        in_specs=[
            pl.BlockSpec(memory_space=pl.ANY),
            pl.BlockSpec(memory_space=pl.ANY),
        ],
        out_specs=pl.BlockSpec(memory_space=pl.ANY),
        scratch_shapes=[
            pltpu.VMEM((K, N), jnp.float32),
            pltpu.VMEM((B * S, K), jnp.float32),
            pltpu.VMEM((B, SH, N), jnp.float32),
            pltpu.VMEM((K, N), jnp.bfloat16),
            pltpu.VMEM((K, N), jnp.bfloat16),
            pltpu.VMEM((B * SH, K), jnp.bfloat16),
            pltpu.VMEM((B * SH, K), jnp.bfloat16),
            pltpu.SemaphoreType.DMA((2,)),
            pltpu.SemaphoreType.DMA((B,)),
            pltpu.SemaphoreType.DMA((2,)),
            pltpu.SemaphoreType.DMA((2, B)),
        ],
        compiler_params=pltpu.CompilerParams(collective_id=0),
    )(O2, Wo)
